# baseline (speedup 1.0000x reference)
import functools

import jax
import jax.numpy as jnp
from jax import lax
from jax.experimental import pallas as pl
from jax.experimental.pallas import tpu as pltpu
from jax.experimental.pallas import tpu_sc as plsc

B, C, H, W = 1, 128, 384, 384
N, K = 10000, 32
HW = H * W

NC = 200
NSTEPS = N // NC
TRB = 512

NCORES, NSUB = 2, 16
NTILES = NCORES * NSUB
RPT = (K * N) // NTILES
GCH = 80

SCH_J, SCH_B = 125, 80
CPT = C // NTILES
ZCH = 16384


def _tr_body(x_ref, o_ref):
    o_ref[...] = x_ref[...].T


def _transpose_cm_to_rm(ft):
    return pl.pallas_call(
        _tr_body,
        grid=(HW // TRB,),
        in_specs=[pl.BlockSpec((C, TRB), lambda i: (0, i))],
        out_specs=pl.BlockSpec((TRB, C), lambda i: (i, 0)),
        out_shape=jax.ShapeDtypeStruct((HW, C), jnp.float32),
    )(ft)


def _transpose_rm_to_cm(x):
    return pl.pallas_call(
        _tr_body,
        grid=(HW // TRB,),
        in_specs=[pl.BlockSpec((TRB, C), lambda i: (i, 0))],
        out_specs=pl.BlockSpec((C, TRB), lambda i: (0, i)),
        out_shape=jax.ShapeDtypeStruct((C, HW), jnp.float32),
    )(x)


def _gather_body(table_hbm, idx_hbm, out_hbm, idx_v, rows_v, sem):
    wid = lax.axis_index("s") * NCORES + lax.axis_index("c")
    base = wid * RPT

    def step(j, carry):
        off = base + j * GCH
        pltpu.sync_copy(idx_hbm.at[pl.ds(off, GCH)], idx_v)
        pltpu.async_copy(table_hbm.at[idx_v], rows_v, sem).wait()
        pltpu.sync_copy(rows_v, out_hbm.at[pl.ds(off, GCH)])
        return carry

    lax.fori_loop(0, RPT // GCH, step, 0)


def _sc_gather(table, idx):
    mesh = plsc.VectorSubcoreMesh(core_axis_name="c", subcore_axis_name="s", num_cores=NCORES, num_subcores=NSUB)
    return pl.kernel(
        _gather_body,
        out_type=jax.ShapeDtypeStruct((K * N, C), jnp.float32),
        mesh=mesh,
        scratch_types=[
            pltpu.VMEM((GCH,), jnp.int32),
            pltpu.VMEM((GCH, C), jnp.float32),
            pltpu.SemaphoreType.DMA,
        ],
    )(table, idx)


def _main_body(d_ref, f_ref, w1b_ref, w2t_ref, b2_ref, wct_ref, bc_ref,
               bnw_ref, bnb_ref, y_ref, aff_ref, acc_ref):
    i = pl.program_id(0)
    d = d_ref[...].reshape(K * NC, 4)
    h1 = jnp.maximum(jnp.dot(d, w1b_ref[...], preferred_element_type=jnp.float32), 0.0)
    wk = jnp.maximum(
        jnp.dot(h1, w2t_ref[...], preferred_element_type=jnp.float32) + b2_ref[...], 0.0)
    prod = wk * f_ref[...].reshape(K * NC, C)
    red = prod.reshape(K, NC, C).sum(axis=0)
    y = jnp.dot(red, wct_ref[...], preferred_element_type=jnp.float32) + bc_ref[...]

    @pl.when(i == 0)
    def _():
        acc_ref[...] = jnp.zeros_like(acc_ref)

    acc_ref[0:1, :] += jnp.sum(y, axis=0, keepdims=True)
    acc_ref[1:2, :] += jnp.sum(y * y, axis=0, keepdims=True)
    y_ref[...] = y

    @pl.when(i == pl.num_programs(0) - 1)
    def _():
        mean = acc_ref[0:1, :] / N
        var = acc_ref[1:2, :] / N - mean * mean
        scale = bnw_ref[...] * lax.rsqrt(var + 1e-5)
        shift = bnb_ref[...] - mean * scale
        aff_ref[...] = jnp.concatenate([scale, shift], axis=0)


def _tc_main(d4, f, w1b, w2t, b2r, wct, bcr, bnwr, bnbr):
    return pl.pallas_call(
        _main_body,
        grid=(NSTEPS,),
        in_specs=[
            pl.BlockSpec((K, NC, 4), lambda i: (0, i, 0)),
            pl.BlockSpec((K, NC, C), lambda i: (0, i, 0)),
            pl.BlockSpec((4, 64), lambda i: (0, 0)),
            pl.BlockSpec((64, C), lambda i: (0, 0)),
            pl.BlockSpec((1, C), lambda i: (0, 0)),
            pl.BlockSpec((C, C), lambda i: (0, 0)),
            pl.BlockSpec((1, C), lambda i: (0, 0)),
            pl.BlockSpec((1, C), lambda i: (0, 0)),
            pl.BlockSpec((1, C), lambda i: (0, 0)),
        ],
        out_specs=[
            pl.BlockSpec((NC, C), lambda i: (i, 0)),
            pl.BlockSpec((2, C), lambda i: (0, 0)),
        ],
        out_shape=[
            jax.ShapeDtypeStruct((N, C), jnp.float32),
            jax.ShapeDtypeStruct((2, C), jnp.float32),
        ],
        scratch_shapes=[pltpu.VMEM((2, C), jnp.float32)],
    )(d4, f, w1b, w2t, b2r, wct, bcr, bnwr, bnbr)


AFB = N


def _aff_body(y_ref, aff_ref, x_ref):
    y = y_ref[...]
    x_ref[...] = jnp.maximum(y * aff_ref[0:1, :] + aff_ref[1:2, :], 0.0)


def _tc_affine_t(y, aff):
    return pl.pallas_call(
        _aff_body,
        grid=(1,),
        in_specs=[
            pl.BlockSpec((AFB, C), lambda i: (0, 0)),
            pl.BlockSpec((2, C), lambda i: (0, 0)),
        ],
        out_specs=pl.BlockSpec((AFB, C), lambda i: (0, 0)),
        out_shape=jax.ShapeDtypeStruct((N, C), jnp.float32),
    )(y, aff)


ZROWS = HW // NSUB
ZB = 128


def _scatter_body(x_hbm, pix_hbm, out_hbm, idx_v, vals_v, zero_v, sem):
    cid = lax.axis_index("c")
    sid = lax.axis_index("s")

    def zfill(j, carry):
        for k in range(C // 16):
            zero_v[j, pl.ds(k * 16, 16)] = jnp.zeros((16,), jnp.float32)
        return carry

    lax.fori_loop(0, ZB, zfill, 0)
    zrows = zero_v

    @pl.when(cid == 0)
    def _():
        def zslice(j, carry):
            pltpu.sync_copy(zrows, out_hbm.at[pl.ds(sid * ZROWS + j * ZB, ZB), :])
            return carry
        lax.fori_loop(0, ZROWS // ZB, zslice, 0)

    plsc.subcore_barrier()

    @pl.when((cid == 0) & (sid == 0))
    def _():
        pltpu.sync_copy(pix_hbm, idx_v)
        for j in range(SCH_J):
            pltpu.sync_copy(x_hbm.at[pl.ds(j * SCH_B, SCH_B), :], vals_v)
            pltpu.async_copy(vals_v, out_hbm.at[idx_v.at[j]], sem).wait()


def _sc_scatter(x, pix3):
    mesh = plsc.VectorSubcoreMesh(core_axis_name="c", subcore_axis_name="s", num_cores=NCORES, num_subcores=NSUB)
    return pl.kernel(
        _scatter_body,
        out_type=jax.ShapeDtypeStruct((HW, C), jnp.float32),
        mesh=mesh,
        scratch_types=[
            pltpu.VMEM((SCH_J, SCH_B), jnp.int32),
            pltpu.VMEM((SCH_B, C), jnp.float32),
            pltpu.VMEM((ZB, C), jnp.float32),
            pltpu.SemaphoreType.DMA,
        ],
    )(x, pix3)


def kernel(feature_tensor, nn_diff_pts_3d, pixel_idxs, nn_pixel_idxs,
           W1, b1, W2, b2, Wc, bc, bn_w, bn_b):
    ft = feature_tensor.reshape(C, HW)
    table = _transpose_cm_to_rm(ft)

    nn_pi = nn_pixel_idxs[0]
    idx_km = (nn_pi[:, :, 1] * W + nn_pi[:, :, 0]).T.reshape(K * N)
    f = _sc_gather(table, idx_km.astype(jnp.int32))

    dd = jnp.transpose(nn_diff_pts_3d[0], (1, 0, 2))
    d4 = jnp.concatenate([dd, jnp.ones((K, N, 1), jnp.float32)], axis=-1)
    w1b = jnp.concatenate([W1.T, b1[None, :]], axis=0)
    y, aff = _tc_main(d4, f.reshape(K, N, C), w1b, W2.T, b2[None, :], Wc.T, bc[None, :],
                      bn_w[None, :], bn_b[None, :])

    x = _tc_affine_t(y, aff)

    pix = (pixel_idxs[0, :, 1] * W + pixel_idxs[0, :, 0]).astype(jnp.int32)
    pix3 = pix.reshape(SCH_J, SCH_B)
    scat = _sc_scatter(x, pix3)
    out = _transpose_rm_to_cm(scat)
    return out.reshape(B, C, H, W)


# speedup vs baseline: 1.5828x; 1.5828x over previous
import jax
import jax.numpy as jnp
from jax import lax
from jax.experimental import pallas as pl
from jax.experimental.pallas import tpu as pltpu
from jax.experimental.pallas import tpu_sc as plsc
from jax._src.pallas import mpmd as _plmpmd

B, C, H, W = 1, 128, 384, 384
N, K = 10000, 32
HW = H * W

NC = 200
NSTEPS = N // NC
TRB = 4608

NCORES, NSUB = 2, 16
NTILES = NCORES * NSUB
RPT = (K * N) // NTILES
GCH = 80
NCH = RPT // GCH

SCB = 128
NSC = N // SCB
NTAIL = N - NSC * SCB


def _mesh():
    return plsc.VectorSubcoreMesh(core_axis_name="c", subcore_axis_name="s",
                                  num_cores=NCORES, num_subcores=NSUB)


def _tr_body(x_ref, o_ref):
    o_ref[...] = x_ref[...].T


def _transpose_cm_to_rm(ft):
    return pl.pallas_call(
        _tr_body,
        grid=(HW // TRB,),
        in_specs=[pl.BlockSpec((C, TRB), lambda i: (0, i))],
        out_specs=pl.BlockSpec((TRB, C), lambda i: (i, 0)),
        out_shape=jax.ShapeDtypeStruct((HW, C), jnp.float32),
    )(ft)


def _transpose_rm_to_cm(x):
    return pl.pallas_call(
        _tr_body,
        grid=(HW // TRB,),
        in_specs=[pl.BlockSpec((TRB, C), lambda i: (i, 0))],
        out_specs=pl.BlockSpec((C, TRB), lambda i: (0, i)),
        out_shape=jax.ShapeDtypeStruct((C, HW), jnp.float32),
    )(x)


def _gather_body(table_hbm, idx3_hbm, out_hbm, idx_v, rows0, rows1,
                 sg0, sg1, sw0, sw1):
    wid = lax.axis_index("s") * NCORES + lax.axis_index("c")
    base = wid * RPT
    pltpu.sync_copy(idx3_hbm.at[wid], idx_v)
    pltpu.async_copy(table_hbm.at[idx_v.at[0]], rows0, sg0)

    def arm(jj, rows_a, sg_a, sw_a, rows_b, sg_b, sw_b):
        pltpu.make_async_copy(table_hbm.at[idx_v.at[jj]], rows_a, sg_a).wait()
        pltpu.async_copy(rows_a, out_hbm.at[pl.ds(base + jj * GCH, GCH)], sw_a)

        @pl.when(jj > 0)
        def _():
            pltpu.make_async_copy(
                rows_b, out_hbm.at[pl.ds(base, GCH)], sw_b).wait()

        @pl.when(jj + 1 < NCH)
        def _():
            pltpu.async_copy(table_hbm.at[idx_v.at[jj + 1]], rows_b, sg_b)

    def chunk(jj, carry):
        @pl.when(jj % 2 == 0)
        def _():
            arm(jj, rows0, sg0, sw0, rows1, sg1, sw1)

        @pl.when(jj % 2 == 1)
        def _():
            arm(jj, rows1, sg1, sw1, rows0, sg0, sw0)

        return carry

    lax.fori_loop(0, NCH, chunk, 0)
    pltpu.make_async_copy(rows0, out_hbm.at[pl.ds(base, GCH)], sw0).wait()


def _sc_gather(table, idx3):
    return pl.kernel(
        _gather_body,
        out_type=jax.ShapeDtypeStruct((N * K, C), jnp.float32),
        mesh=_mesh(),
        scratch_types=[
            pltpu.VMEM((NCH, GCH), jnp.int32),
            pltpu.VMEM((GCH, C), jnp.float32),
            pltpu.VMEM((GCH, C), jnp.float32),
            pltpu.SemaphoreType.DMA,
            pltpu.SemaphoreType.DMA,
            pltpu.SemaphoreType.DMA,
            pltpu.SemaphoreType.DMA,
        ],
    )(table, idx3)


def _main_body(d_ref, f_ref, w1b_ref, w2t_ref, b2_ref, wct_ref, bc_ref,
               bnw_ref, bnb_ref, y_ref, aff_ref, acc_ref):
    i = pl.program_id(0)
    d = d_ref[...]
    h1 = jnp.maximum(jnp.dot(d, w1b_ref[...], preferred_element_type=jnp.float32), 0.0)
    wk = jnp.maximum(
        jnp.dot(h1, w2t_ref[...], preferred_element_type=jnp.float32) + b2_ref[...], 0.0)
    prod = wk * f_ref[...]
    red = prod.reshape(NC, K, C).sum(axis=1)
    y = jnp.dot(red, wct_ref[...], preferred_element_type=jnp.float32) + bc_ref[...]

    @pl.when(i == 0)
    def _():
        acc_ref[...] = jnp.zeros_like(acc_ref)

    acc_ref[0:1, :] += jnp.sum(y, axis=0, keepdims=True)
    acc_ref[1:2, :] += jnp.sum(y * y, axis=0, keepdims=True)
    y_ref[...] = y

    @pl.when(i == pl.num_programs(0) - 1)
    def _():
        mean = acc_ref[0:1, :] / N
        var = acc_ref[1:2, :] / N - mean * mean
        scale = bnw_ref[...] * lax.rsqrt(var + 1e-5)
        shift = bnb_ref[...] - mean * scale
        aff_ref[...] = jnp.concatenate([scale, shift], axis=0)


def _tc_main(d4, f, w1b, w2t, b2r, wct, bcr, bnwr, bnbr):
    return pl.pallas_call(
        _main_body,
        grid=(NSTEPS,),
        in_specs=[
            pl.BlockSpec((NC * K, 4), lambda i: (i, 0)),
            pl.BlockSpec((NC * K, C), lambda i: (i, 0)),
            pl.BlockSpec((4, 64), lambda i: (0, 0)),
            pl.BlockSpec((64, C), lambda i: (0, 0)),
            pl.BlockSpec((1, C), lambda i: (0, 0)),
            pl.BlockSpec((C, C), lambda i: (0, 0)),
            pl.BlockSpec((1, C), lambda i: (0, 0)),
            pl.BlockSpec((1, C), lambda i: (0, 0)),
            pl.BlockSpec((1, C), lambda i: (0, 0)),
        ],
        out_specs=[
            pl.BlockSpec((NC, C), lambda i: (i, 0)),
            pl.BlockSpec((2, C), lambda i: (0, 0)),
        ],
        out_shape=[
            jax.ShapeDtypeStruct((N, C), jnp.float32),
            jax.ShapeDtypeStruct((2, C), jnp.float32),
        ],
        scratch_shapes=[pltpu.VMEM((2, C), jnp.float32)],
    )(d4, f, w1b, w2t, b2r, wct, bcr, bnwr, bnbr)


def _aff_body(y_ref, aff_ref, x_ref):
    y = y_ref[...]
    x_ref[...] = jnp.maximum(y * aff_ref[0:1, :] + aff_ref[1:2, :], 0.0)


def _tc_affine(y, aff):
    return pl.pallas_call(
        _aff_body,
        grid=(1,),
        in_specs=[
            pl.BlockSpec((N, C), lambda i: (0, 0)),
            pl.BlockSpec((2, C), lambda i: (0, 0)),
        ],
        out_specs=pl.BlockSpec((N, C), lambda i: (0, 0)),
        out_shape=jax.ShapeDtypeStruct((N, C), jnp.float32),
    )(y, aff)


def _scatter_body(x_hbm, pixm_hbm, pixt_hbm, base_hbm, out_hbm,
                  idx_v, vals0, vals1, tidx, tvals, sv0, sv1, ss):
    cid = lax.axis_index("c")
    sid = lax.axis_index("s")

    @pl.when((cid == 0) & (sid == 0))
    def _():
        pltpu.sync_copy(pixm_hbm, idx_v)
        pltpu.async_copy(x_hbm.at[pl.ds(0, SCB)], vals0, sv0)

        def arm(jj, vals_a, sv_a, vals_b, sv_b):
            pltpu.make_async_copy(
                x_hbm.at[pl.ds(0, SCB)], vals_a, sv_a).wait()

            @pl.when(jj + 1 < NSC)
            def _():
                pltpu.async_copy(
                    x_hbm.at[pl.ds((jj + 1) * SCB, SCB)], vals_b, sv_b)

            pltpu.async_copy(vals_a, out_hbm.at[idx_v.at[jj]], ss).wait()

        def chunk(jj, carry):
            @pl.when(jj % 2 == 0)
            def _():
                arm(jj, vals0, sv0, vals1, sv1)

            @pl.when(jj % 2 == 1)
            def _():
                arm(jj, vals1, sv1, vals0, sv0)

            return carry

        lax.fori_loop(0, NSC, chunk, 0)

        pltpu.sync_copy(pixt_hbm, tidx)
        pltpu.sync_copy(x_hbm.at[pl.ds(NSC * SCB, NTAIL)], tvals)
        pltpu.async_copy(tvals, out_hbm.at[tidx], ss).wait()


def _sc_scatter(x, pix_main, pix_tail, base):
    fn = _plmpmd._mpmd_map(
        ((_mesh(), _scatter_body),),
        jax.ShapeDtypeStruct((HW, C), jnp.float32),
        input_output_aliases={3: 0},
        scratch_types=[
            pltpu.VMEM((NSC, SCB), jnp.int32),
            pltpu.VMEM((SCB, C), jnp.float32),
            pltpu.VMEM((SCB, C), jnp.float32),
            pltpu.VMEM((NTAIL,), jnp.int32),
            pltpu.VMEM((NTAIL, C), jnp.float32),
            pltpu.SemaphoreType.DMA,
            pltpu.SemaphoreType.DMA,
            pltpu.SemaphoreType.DMA,
        ],
    )
    return fn(x, pix_main, pix_tail, base)


def kernel(feature_tensor, nn_diff_pts_3d, pixel_idxs, nn_pixel_idxs,
           W1, b1, W2, b2, Wc, bc, bn_w, bn_b):
    ft = feature_tensor.reshape(C, HW)
    table = _transpose_cm_to_rm(ft)

    nn_pi = nn_pixel_idxs[0]
    idx = (nn_pi[:, :, 1] * W + nn_pi[:, :, 0]).astype(jnp.int32)
    f = _sc_gather(table, idx.reshape(NTILES, NCH, GCH))

    d4 = jnp.concatenate(
        [nn_diff_pts_3d[0].reshape(N * K, 3),
         jnp.ones((N * K, 1), jnp.float32)], axis=-1)
    w1b = jnp.concatenate([W1.T, b1[None, :]], axis=0)
    y, aff = _tc_main(d4, f, w1b, W2.T, b2[None, :], Wc.T, bc[None, :],
                      bn_w[None, :], bn_b[None, :])

    x = _tc_affine(y, aff)

    pix = (pixel_idxs[0, :, 1] * W + pixel_idxs[0, :, 0]).astype(jnp.int32)
    base = jnp.zeros((HW, C), jnp.float32)
    scat = _sc_scatter(x, pix[:NSC * SCB].reshape(NSC, SCB),
                       pix[NSC * SCB:], base)
    out = _transpose_rm_to_cm(scat)
    return out.reshape(B, C, H, W)
